# baseline (speedup 1.0000x reference)
import math

import jax
import jax.numpy as jnp
from jax import lax
from jax.experimental import pallas as pl
from jax.experimental.pallas import tpu as pltpu
from jax.experimental.pallas import tpu_sc as plsc

N_LEVELS = 16
FDIM = 2
TSZ = 1 << 19
BASE_RES = 16
PER_LEVEL_SCALE = 1.5
P1 = 2654435761
P2 = 805459861

NC = 2
NS = 16
NW = NC * NS
LANES = 16

GRP = TSZ // 4
RES = [int(math.floor(BASE_RES * PER_LEVEL_SCALE ** l)) for l in range(N_LEVELS)]
DENSE = [(r + 1) ** 3 <= TSZ for r in RES]
CORNERS = [(cx, cy, cz) for cz in (0, 1) for cy in (0, 1) for cx in (0, 1)]

C = 128


def _body(xn, tab, out, coords_v, frac_v,
          g0, g1, g2, g3, g4, g5, g6, g7,
          o0, o1, o2, o3, o4, o5, o6, o7,
          r0, r1, r2, r3, r4, r5, r6, r7, obuf_v, sem):
    grp_vs = (g0, g1, g2, g3, g4, g5, g6, g7)
    off_vs = (o0, o1, o2, o3, o4, o5, o6, o7)
    rows_vs = (r0, r1, r2, r3, r4, r5, r6, r7)
    n = out.shape[0]
    ppt = n // NW
    nchunk = ppt // C
    wid = lax.axis_index("s") * NC + lax.axis_index("c")
    lane = jnp.arange(LANES, dtype=jnp.int32)
    zeros16 = jnp.zeros((LANES,), jnp.int32)
    ones16 = jnp.full((LANES,), 1, jnp.int32)
    twos16 = jnp.full((LANES,), 2, jnp.int32)

    def chunk_body(g, carry):
        base = wid * ppt + g * C
        pltpu.sync_copy(xn.at[pl.ds(base, C)], coords_v)

        for l in range(N_LEVELS):
            res = RES[l]

            def pa(i, c_, l=l, res=res):
                o = i * LANES
                rows16 = o + lane
                x = plsc.load_gather(coords_v, [rows16, zeros16])
                y = plsc.load_gather(coords_v, [rows16, ones16])
                z = plsc.load_gather(coords_v, [rows16, twos16])
                px = x * float(res)
                py = y * float(res)
                pz = z * float(res)
                ix = px.astype(jnp.int32)
                iy = py.astype(jnp.int32)
                iz = pz.astype(jnp.int32)
                plsc.store_scatter(frac_v, [zeros16, rows16],
                                   px - ix.astype(jnp.float32))
                plsc.store_scatter(frac_v, [ones16, rows16],
                                   py - iy.astype(jnp.float32))
                plsc.store_scatter(frac_v, [twos16, rows16],
                                   pz - iz.astype(jnp.float32))
                if DENSE[l]:
                    s = res + 1
                    base_d = ix + s * (iy + s * iz)
                    for ci, (cx, cy, cz) in enumerate(CORNERS):
                        idx = base_d + (cx + s * cy + s * s * cz)
                        plsc.store_scatter(
                            grp_vs[ci], [rows16],
                            (idx >> 2) + l * GRP)
                        plsc.store_scatter(
                            off_vs[ci], [rows16], (idx & 3) << 1)
                else:
                    ux = ix.astype(jnp.uint32)
                    uy = iy.astype(jnp.uint32) * jnp.uint32(P1)
                    uz = iz.astype(jnp.uint32) * jnp.uint32(P2)
                    ux1 = ux + jnp.uint32(1)
                    uy1 = uy + jnp.uint32(P1)
                    uz1 = uz + jnp.uint32(P2)
                    for ci, (cx, cy, cz) in enumerate(CORNERS):
                        h = (ux1 if cx else ux) ^ (uy1 if cy else uy) ^ (uz1 if cz else uz)
                        idx = (h & jnp.uint32(TSZ - 1)).astype(jnp.int32)
                        plsc.store_scatter(
                            grp_vs[ci], [rows16], (idx >> 2) + l * GRP)
                        plsc.store_scatter(
                            off_vs[ci], [rows16], (idx & 3) << 1)
                return c_

            lax.fori_loop(0, C // LANES, pa, 0)

            copies = [
                pltpu.async_copy(tab.at[grp_vs[ci]], rows_vs[ci], sem)
                for ci in range(8)
            ]
            for cp in copies:
                cp.wait()

            def pc(i, c_, l=l):
                o = i * LANES
                rows16 = o + lane
                fx = plsc.load_gather(frac_v, [zeros16, rows16])
                fy = plsc.load_gather(frac_v, [ones16, rows16])
                fz = plsc.load_gather(frac_v, [twos16, rows16])
                gx = 1.0 - fx
                gy = 1.0 - fy
                gz = 1.0 - fz
                wxy = {(0, 0): gx * gy, (1, 0): fx * gy,
                       (0, 1): gx * fy, (1, 1): fx * fy}
                acc0 = jnp.zeros((LANES,), jnp.float32)
                acc1 = jnp.zeros((LANES,), jnp.float32)
                for ci, (cx, cy, cz) in enumerate(CORNERS):
                    w = wxy[(cx, cy)] * (fz if cz else gz)
                    lo = plsc.load_gather(off_vs[ci], [rows16])
                    v0 = plsc.load_gather(rows_vs[ci], [rows16, lo])
                    v1 = plsc.load_gather(rows_vs[ci], [rows16, lo + 1])
                    acc0 = acc0 + v0 * w
                    acc1 = acc1 + v1 * w
                plsc.store_scatter(
                    obuf_v, [rows16, jnp.full((LANES,), 2 * l, jnp.int32)], acc0)
                plsc.store_scatter(
                    obuf_v, [rows16, jnp.full((LANES,), 2 * l + 1, jnp.int32)], acc1)
                return c_

            lax.fori_loop(0, C // LANES, pc, 0)

        pltpu.sync_copy(obuf_v, out.at[pl.ds(base, C)])
        return carry

    lax.fori_loop(0, nchunk, chunk_body, 0)


def kernel(xc, table, bbox_min, bbox_max):
    n = xc.shape[0]
    assert n % (NW * C) == 0
    scale = jnp.clip(bbox_max - bbox_min, 1e-6, None)
    xn = (xc - bbox_min[None, :]) / scale[None, :]
    tab = table.reshape(N_LEVELS * GRP, 4 * FDIM)

    mesh = plsc.VectorSubcoreMesh(core_axis_name="c", subcore_axis_name="s")
    f = pl.kernel(
        _body,
        out_type=jax.ShapeDtypeStruct((n, N_LEVELS * FDIM), jnp.float32),
        mesh=mesh,
        compiler_params=pltpu.CompilerParams(
            needs_layout_passes=False, use_tc_tiling_on_sc=False),
        scratch_types=[
            pltpu.VMEM((C, 3), jnp.float32),
            pltpu.VMEM((3, C), jnp.float32),
            *[pltpu.VMEM((C,), jnp.int32) for _ in range(8)],
            *[pltpu.VMEM((C,), jnp.int32) for _ in range(8)],
            *[pltpu.VMEM((C, 4 * FDIM), jnp.float32) for _ in range(8)],
            pltpu.VMEM((C, N_LEVELS * FDIM), jnp.float32),
            pltpu.SemaphoreType.DMA,
        ],
    )
    return f(xn, tab)


# speedup vs baseline: 1.0644x; 1.0644x over previous
import math

import jax
import jax.numpy as jnp
from jax import lax
from jax.experimental import pallas as pl
from jax.experimental.pallas import tpu as pltpu
from jax.experimental.pallas import tpu_sc as plsc

N_LEVELS = 16
FDIM = 2
TSZ = 1 << 19
BASE_RES = 16
PER_LEVEL_SCALE = 1.5
P1 = 2654435761
P2 = 805459861

NC = 2
NS = 16
NW = NC * NS
LANES = 16

GRP = TSZ // 4
RES = [int(math.floor(BASE_RES * PER_LEVEL_SCALE ** l)) for l in range(N_LEVELS)]
DENSE = [(r + 1) ** 3 <= TSZ for r in RES]
CORNERS = [(cx, cy, cz) for cz in (0, 1) for cy in (0, 1) for cx in (0, 1)]

C = 512


def _body(xn, tab, out, coords_v, frac_v,
          g0, g1, g2, g3, g4, g5, g6, g7,
          o0, o1, o2, o3, o4, o5, o6, o7,
          r0, r1, r2, r3, r4, r5, r6, r7, obuf_v, sem):
    grp_vs = (g0, g1, g2, g3, g4, g5, g6, g7)
    off_vs = (o0, o1, o2, o3, o4, o5, o6, o7)
    rows_vs = (r0, r1, r2, r3, r4, r5, r6, r7)
    n = out.shape[0]
    ppt = n // NW
    nchunk = ppt // C
    wid = lax.axis_index("s") * NC + lax.axis_index("c")
    lane = jnp.arange(LANES, dtype=jnp.int32)
    zeros16 = jnp.zeros((LANES,), jnp.int32)
    ones16 = jnp.full((LANES,), 1, jnp.int32)
    twos16 = jnp.full((LANES,), 2, jnp.int32)

    def chunk_body(g, carry):
        base = wid * ppt + g * C
        pltpu.sync_copy(xn.at[pl.ds(base, C)], coords_v)

        for l in range(N_LEVELS):
            res = RES[l]

            def pa(i, c_, l=l, res=res):
                o = i * LANES
                rows16 = o + lane
                x = plsc.load_gather(coords_v, [rows16, zeros16])
                y = plsc.load_gather(coords_v, [rows16, ones16])
                z = plsc.load_gather(coords_v, [rows16, twos16])
                px = x * float(res)
                py = y * float(res)
                pz = z * float(res)
                ix = px.astype(jnp.int32)
                iy = py.astype(jnp.int32)
                iz = pz.astype(jnp.int32)
                plsc.store_scatter(frac_v, [zeros16, rows16],
                                   px - ix.astype(jnp.float32))
                plsc.store_scatter(frac_v, [ones16, rows16],
                                   py - iy.astype(jnp.float32))
                plsc.store_scatter(frac_v, [twos16, rows16],
                                   pz - iz.astype(jnp.float32))
                if DENSE[l]:
                    s = res + 1
                    base_d = ix + s * (iy + s * iz)
                    for ci, (cx, cy, cz) in enumerate(CORNERS):
                        idx = base_d + (cx + s * cy + s * s * cz)
                        plsc.store_scatter(
                            grp_vs[ci], [rows16],
                            (idx >> 2) + l * GRP)
                        plsc.store_scatter(
                            off_vs[ci], [rows16], (idx & 3) << 1)
                else:
                    ux = ix.astype(jnp.uint32)
                    uy = iy.astype(jnp.uint32) * jnp.uint32(P1)
                    uz = iz.astype(jnp.uint32) * jnp.uint32(P2)
                    ux1 = ux + jnp.uint32(1)
                    uy1 = uy + jnp.uint32(P1)
                    uz1 = uz + jnp.uint32(P2)
                    for ci, (cx, cy, cz) in enumerate(CORNERS):
                        h = (ux1 if cx else ux) ^ (uy1 if cy else uy) ^ (uz1 if cz else uz)
                        idx = (h & jnp.uint32(TSZ - 1)).astype(jnp.int32)
                        plsc.store_scatter(
                            grp_vs[ci], [rows16], (idx >> 2) + l * GRP)
                        plsc.store_scatter(
                            off_vs[ci], [rows16], (idx & 3) << 1)
                return c_

            lax.fori_loop(0, C // LANES, pa, 0)

            copies = [
                pltpu.async_copy(tab.at[grp_vs[ci]], rows_vs[ci], sem)
                for ci in range(8)
            ]
            for cp in copies:
                cp.wait()

            def pc(i, c_, l=l):
                o = i * LANES
                rows16 = o + lane
                fx = plsc.load_gather(frac_v, [zeros16, rows16])
                fy = plsc.load_gather(frac_v, [ones16, rows16])
                fz = plsc.load_gather(frac_v, [twos16, rows16])
                gx = 1.0 - fx
                gy = 1.0 - fy
                gz = 1.0 - fz
                wxy = {(0, 0): gx * gy, (1, 0): fx * gy,
                       (0, 1): gx * fy, (1, 1): fx * fy}
                acc0 = jnp.zeros((LANES,), jnp.float32)
                acc1 = jnp.zeros((LANES,), jnp.float32)
                for ci, (cx, cy, cz) in enumerate(CORNERS):
                    w = wxy[(cx, cy)] * (fz if cz else gz)
                    lo = plsc.load_gather(off_vs[ci], [rows16])
                    v0 = plsc.load_gather(rows_vs[ci], [rows16, lo])
                    v1 = plsc.load_gather(rows_vs[ci], [rows16, lo + 1])
                    acc0 = acc0 + v0 * w
                    acc1 = acc1 + v1 * w
                plsc.store_scatter(
                    obuf_v, [rows16, jnp.full((LANES,), 2 * l, jnp.int32)], acc0)
                plsc.store_scatter(
                    obuf_v, [rows16, jnp.full((LANES,), 2 * l + 1, jnp.int32)], acc1)
                return c_

            lax.fori_loop(0, C // LANES, pc, 0)

        pltpu.sync_copy(obuf_v, out.at[pl.ds(base, C)])
        return carry

    lax.fori_loop(0, nchunk, chunk_body, 0)


def kernel(xc, table, bbox_min, bbox_max):
    n = xc.shape[0]
    assert n % (NW * C) == 0
    scale = jnp.clip(bbox_max - bbox_min, 1e-6, None)
    xn = (xc - bbox_min[None, :]) / scale[None, :]
    tab = table.reshape(N_LEVELS * GRP, 4 * FDIM)

    mesh = plsc.VectorSubcoreMesh(core_axis_name="c", subcore_axis_name="s")
    f = pl.kernel(
        _body,
        out_type=jax.ShapeDtypeStruct((n, N_LEVELS * FDIM), jnp.float32),
        mesh=mesh,
        compiler_params=pltpu.CompilerParams(
            needs_layout_passes=False, use_tc_tiling_on_sc=False),
        scratch_types=[
            pltpu.VMEM((C, 3), jnp.float32),
            pltpu.VMEM((3, C), jnp.float32),
            *[pltpu.VMEM((C,), jnp.int32) for _ in range(8)],
            *[pltpu.VMEM((C,), jnp.int32) for _ in range(8)],
            *[pltpu.VMEM((C, 4 * FDIM), jnp.float32) for _ in range(8)],
            pltpu.VMEM((C, N_LEVELS * FDIM), jnp.float32),
            pltpu.SemaphoreType.DMA,
        ],
    )
    return f(xn, tab)


# speedup vs baseline: 2.6914x; 2.5284x over previous
import math

import jax
import jax.numpy as jnp
from jax import lax
from jax.experimental import pallas as pl
from jax.experimental.pallas import tpu as pltpu
from jax.experimental.pallas import tpu_sc as plsc

N_LEVELS = 16
FDIM = 2
TSZ = 1 << 19
BASE_RES = 16
PER_LEVEL_SCALE = 1.5
P1 = 2654435761
P2 = 805459861

NC = 2
NS = 16
NW = NC * NS
LANES = 16

GRP = TSZ // 4
RES = [int(math.floor(BASE_RES * PER_LEVEL_SCALE ** l)) for l in range(N_LEVELS)]
DENSE = [(r + 1) ** 3 <= TSZ for r in RES]
CORNERS = [(cx, cy, cz) for cz in (0, 1) for cy in (0, 1) for cx in (0, 1)]

C = 512


def _body(xn, tab, out, coords_v, frac_v,
          g0, g1, g2, g3, g4, g5, g6, g7,
          o0, o1, o2, o3, o4, o5, o6, o7,
          r0, r1, r2, r3, r4, r5, r6, r7, obuf_v, sem):
    grp_vs = (g0, g1, g2, g3, g4, g5, g6, g7)
    off_vs = (o0, o1, o2, o3, o4, o5, o6, o7)
    rows_vs = (r0, r1, r2, r3, r4, r5, r6, r7)
    n = out.shape[0]
    ppt = n // NW
    nchunk = ppt // C
    wid = lax.axis_index("s") * NC + lax.axis_index("c")
    lane = jnp.arange(LANES, dtype=jnp.int32)
    zeros16 = jnp.zeros((LANES,), jnp.int32)
    ones16 = jnp.full((LANES,), 1, jnp.int32)
    twos16 = jnp.full((LANES,), 2, jnp.int32)

    def chunk_body(g, carry):
        base = wid * ppt + g * C
        pltpu.sync_copy(xn.at[pl.ds(base, C)], coords_v)

        for l in range(N_LEVELS):
            res = RES[l]

            def pa(i, c_, l=l, res=res):
                o = i * LANES
                rows16 = o + lane
                x = plsc.load_gather(coords_v, [rows16, zeros16])
                y = plsc.load_gather(coords_v, [rows16, ones16])
                z = plsc.load_gather(coords_v, [rows16, twos16])
                px = x * float(res)
                py = y * float(res)
                pz = z * float(res)
                ix = px.astype(jnp.int32)
                iy = py.astype(jnp.int32)
                iz = pz.astype(jnp.int32)
                plsc.store_scatter(frac_v, [zeros16, rows16],
                                   px - ix.astype(jnp.float32))
                plsc.store_scatter(frac_v, [ones16, rows16],
                                   py - iy.astype(jnp.float32))
                plsc.store_scatter(frac_v, [twos16, rows16],
                                   pz - iz.astype(jnp.float32))
                if DENSE[l]:
                    s = res + 1
                    base_d = ix + s * (iy + s * iz)
                    for ci, (cx, cy, cz) in enumerate(CORNERS):
                        idx = base_d + (cx + s * cy + s * s * cz)
                        plsc.store_scatter(
                            grp_vs[ci], [rows16],
                            (idx >> 2) + l * GRP)
                        plsc.store_scatter(
                            off_vs[ci], [rows16], (idx & 3) << 1)
                else:
                    ux = ix.astype(jnp.uint32)
                    uy = iy.astype(jnp.uint32) * jnp.uint32(P1)
                    uz = iz.astype(jnp.uint32) * jnp.uint32(P2)
                    ux1 = ux + jnp.uint32(1)
                    uy1 = uy + jnp.uint32(P1)
                    uz1 = uz + jnp.uint32(P2)
                    for ci, (cx, cy, cz) in enumerate(CORNERS):
                        h = (ux1 if cx else ux) ^ (uy1 if cy else uy) ^ (uz1 if cz else uz)
                        idx = (h & jnp.uint32(TSZ - 1)).astype(jnp.int32)
                        plsc.store_scatter(
                            grp_vs[ci], [rows16], (idx >> 2) + l * GRP)
                        plsc.store_scatter(
                            off_vs[ci], [rows16], (idx & 3) << 1)
                return c_

            lax.fori_loop(0, C // LANES, pa, 0)

            copies = [
                pltpu.async_copy(tab.at[grp_vs[ci]], rows_vs[ci], sem)
                for ci in range(8)
            ]
            for cp in copies:
                cp.wait()

            def pc(i, c_, l=l):
                o = i * LANES
                rows16 = o + lane
                fx = plsc.load_gather(frac_v, [zeros16, rows16])
                fy = plsc.load_gather(frac_v, [ones16, rows16])
                fz = plsc.load_gather(frac_v, [twos16, rows16])
                gx = 1.0 - fx
                gy = 1.0 - fy
                gz = 1.0 - fz
                wxy = {(0, 0): gx * gy, (1, 0): fx * gy,
                       (0, 1): gx * fy, (1, 1): fx * fy}
                acc0 = jnp.zeros((LANES,), jnp.float32)
                acc1 = jnp.zeros((LANES,), jnp.float32)
                for ci, (cx, cy, cz) in enumerate(CORNERS):
                    w = wxy[(cx, cy)] * (fz if cz else gz)
                    lo = plsc.load_gather(off_vs[ci], [rows16])
                    v0 = plsc.load_gather(rows_vs[ci], [rows16, lo])
                    v1 = plsc.load_gather(rows_vs[ci], [rows16, lo + 1])
                    acc0 = acc0 + v0 * w
                    acc1 = acc1 + v1 * w
                plsc.store_scatter(
                    obuf_v, [rows16, jnp.full((LANES,), 2 * l, jnp.int32)], acc0)
                plsc.store_scatter(
                    obuf_v, [rows16, jnp.full((LANES,), 2 * l + 1, jnp.int32)], acc1)
                return c_

            lax.fori_loop(0, C // LANES, pc, 0)

        pltpu.sync_copy(obuf_v, out.at[pl.ds(base, C)])
        return carry

    lax.fori_loop(0, nchunk, chunk_body, 0)


def kernel(xc, table, bbox_min, bbox_max):
    n = xc.shape[0]
    assert n % (NW * C) == 0
    scale = jnp.clip(bbox_max - bbox_min, 1e-6, None)
    xn = (xc - bbox_min[None, :]) / scale[None, :]
    tabT = jnp.swapaxes(table, 1, 2).reshape(N_LEVELS, FDIM, GRP, 4)
    tab = jnp.transpose(tabT, (0, 2, 3, 1)).reshape(N_LEVELS * GRP, 4 * FDIM)

    mesh = plsc.VectorSubcoreMesh(core_axis_name="c", subcore_axis_name="s")
    f = pl.kernel(
        _body,
        out_type=jax.ShapeDtypeStruct((n, N_LEVELS * FDIM), jnp.float32),
        mesh=mesh,
        compiler_params=pltpu.CompilerParams(
            needs_layout_passes=False, use_tc_tiling_on_sc=False),
        scratch_types=[
            pltpu.VMEM((C, 3), jnp.float32),
            pltpu.VMEM((3, C), jnp.float32),
            *[pltpu.VMEM((C,), jnp.int32) for _ in range(8)],
            *[pltpu.VMEM((C,), jnp.int32) for _ in range(8)],
            *[pltpu.VMEM((C, 4 * FDIM), jnp.float32) for _ in range(8)],
            pltpu.VMEM((C, N_LEVELS * FDIM), jnp.float32),
            pltpu.SemaphoreType.DMA,
        ],
    )
    return f(xn, tab)


# speedup vs baseline: 3.4909x; 1.2971x over previous
import math

import jax
import jax.numpy as jnp
from jax import lax
from jax.experimental import pallas as pl
from jax.experimental.pallas import tpu as pltpu
from jax.experimental.pallas import tpu_sc as plsc

N_LEVELS = 16
FDIM = 2
TSZ = 1 << 19
BASE_RES = 16
PER_LEVEL_SCALE = 1.5
P1 = 2654435761
P2 = 805459861

NC = 2
NS = 16
NW = NC * NS
LANES = 16

GRP = TSZ // 4
RES = [int(math.floor(BASE_RES * PER_LEVEL_SCALE ** l)) for l in range(N_LEVELS)]
DENSE = [(r + 1) ** 3 <= TSZ for r in RES]
CORNERS = [(cx, cy, cz) for cz in (0, 1) for cy in (0, 1) for cx in (0, 1)]

C = 512


def _body(xn, tab, out, coords_v, frac_v, *rest):
    grp_vs = (rest[0:8], rest[8:16])
    off_vs = (rest[16:24], rest[24:32])
    rows_vs = (rest[32:40], rest[40:48])
    obuf_v = rest[48]
    sems = (rest[49], rest[50])
    n = out.shape[0]
    ppt = n // NW
    nchunk = ppt // C
    wid = lax.axis_index("s") * NC + lax.axis_index("c")
    lane = jnp.arange(LANES, dtype=jnp.int32)
    zeros16 = jnp.zeros((LANES,), jnp.int32)
    ones16 = jnp.full((LANES,), 1, jnp.int32)
    twos16 = jnp.full((LANES,), 2, jnp.int32)

    def a_and_fire(l, par):
        res = RES[l]
        f0 = 3 * par

        def pa(i, c_, l=l, res=res, par=par, f0=f0):
            o = i * LANES
            rows16 = o + lane
            x = plsc.load_gather(coords_v, [rows16, zeros16])
            y = plsc.load_gather(coords_v, [rows16, ones16])
            z = plsc.load_gather(coords_v, [rows16, twos16])
            px = x * float(res)
            py = y * float(res)
            pz = z * float(res)
            ix = px.astype(jnp.int32)
            iy = py.astype(jnp.int32)
            iz = pz.astype(jnp.int32)
            plsc.store_scatter(frac_v, [jnp.full((LANES,), f0, jnp.int32), rows16],
                               px - ix.astype(jnp.float32))
            plsc.store_scatter(frac_v, [jnp.full((LANES,), f0 + 1, jnp.int32), rows16],
                               py - iy.astype(jnp.float32))
            plsc.store_scatter(frac_v, [jnp.full((LANES,), f0 + 2, jnp.int32), rows16],
                               pz - iz.astype(jnp.float32))
            if DENSE[l]:
                s = res + 1
                base_d = ix + s * (iy + s * iz)
                for ci, (cx, cy, cz) in enumerate(CORNERS):
                    idx = base_d + (cx + s * cy + s * s * cz)
                    plsc.store_scatter(grp_vs[par][ci], [rows16],
                                       (idx >> 2) + l * GRP)
                    plsc.store_scatter(off_vs[par][ci], [rows16], (idx & 3) << 1)
            else:
                ux = ix.astype(jnp.uint32)
                uy = iy.astype(jnp.uint32) * jnp.uint32(P1)
                uz = iz.astype(jnp.uint32) * jnp.uint32(P2)
                ux1 = ux + jnp.uint32(1)
                uy1 = uy + jnp.uint32(P1)
                uz1 = uz + jnp.uint32(P2)
                for ci, (cx, cy, cz) in enumerate(CORNERS):
                    h = (ux1 if cx else ux) ^ (uy1 if cy else uy) ^ (uz1 if cz else uz)
                    idx = (h & jnp.uint32(TSZ - 1)).astype(jnp.int32)
                    plsc.store_scatter(grp_vs[par][ci], [rows16],
                                       (idx >> 2) + l * GRP)
                    plsc.store_scatter(off_vs[par][ci], [rows16], (idx & 3) << 1)
            return c_

        lax.fori_loop(0, C // LANES, pa, 0)
        return [
            pltpu.async_copy(tab.at[grp_vs[par][ci]], rows_vs[par][ci], sems[par])
            for ci in range(8)
        ]

    def interp(l, par):
        f0 = 3 * par

        def pc(i, c_, l=l, par=par, f0=f0):
            o = i * LANES
            rows16 = o + lane
            fx = plsc.load_gather(frac_v, [jnp.full((LANES,), f0, jnp.int32), rows16])
            fy = plsc.load_gather(frac_v, [jnp.full((LANES,), f0 + 1, jnp.int32), rows16])
            fz = plsc.load_gather(frac_v, [jnp.full((LANES,), f0 + 2, jnp.int32), rows16])
            gx = 1.0 - fx
            gy = 1.0 - fy
            gz = 1.0 - fz
            wxy = {(0, 0): gx * gy, (1, 0): fx * gy,
                   (0, 1): gx * fy, (1, 1): fx * fy}
            acc0 = jnp.zeros((LANES,), jnp.float32)
            acc1 = jnp.zeros((LANES,), jnp.float32)
            for ci, (cx, cy, cz) in enumerate(CORNERS):
                w = wxy[(cx, cy)] * (fz if cz else gz)
                lo = plsc.load_gather(off_vs[par][ci], [rows16])
                v0 = plsc.load_gather(rows_vs[par][ci], [rows16, lo])
                v1 = plsc.load_gather(rows_vs[par][ci], [rows16, lo + 1])
                acc0 = acc0 + v0 * w
                acc1 = acc1 + v1 * w
            plsc.store_scatter(
                obuf_v, [rows16, jnp.full((LANES,), 2 * l, jnp.int32)], acc0)
            plsc.store_scatter(
                obuf_v, [rows16, jnp.full((LANES,), 2 * l + 1, jnp.int32)], acc1)
            return c_

        lax.fori_loop(0, C // LANES, pc, 0)

    def chunk_body(g, carry):
        base = wid * ppt + g * C
        pltpu.sync_copy(xn.at[pl.ds(base, C)], coords_v)

        copies = a_and_fire(0, 0)
        for l in range(N_LEVELS):
            par = l % 2
            nxt = a_and_fire(l + 1, 1 - par) if l + 1 < N_LEVELS else None
            for cp in copies:
                cp.wait()
            interp(l, par)
            copies = nxt

        pltpu.sync_copy(obuf_v, out.at[pl.ds(base, C)])
        return carry

    lax.fori_loop(0, nchunk, chunk_body, 0)


def kernel(xc, table, bbox_min, bbox_max):
    n = xc.shape[0]
    assert n % (NW * C) == 0
    scale = jnp.clip(bbox_max - bbox_min, 1e-6, None)
    xn = (xc - bbox_min[None, :]) / scale[None, :]
    tabT = jnp.swapaxes(table, 1, 2).reshape(N_LEVELS, FDIM, GRP, 4)
    tab = jnp.transpose(tabT, (0, 2, 3, 1)).reshape(N_LEVELS * GRP, 4 * FDIM)

    mesh = plsc.VectorSubcoreMesh(core_axis_name="c", subcore_axis_name="s")
    f = pl.kernel(
        _body,
        out_type=jax.ShapeDtypeStruct((n, N_LEVELS * FDIM), jnp.float32),
        mesh=mesh,
        compiler_params=pltpu.CompilerParams(
            needs_layout_passes=False, use_tc_tiling_on_sc=False),
        scratch_types=[
            pltpu.VMEM((C, 3), jnp.float32),
            pltpu.VMEM((6, C), jnp.float32),
            *[pltpu.VMEM((C,), jnp.int32) for _ in range(16)],
            *[pltpu.VMEM((C,), jnp.int32) for _ in range(16)],
            *[pltpu.VMEM((C, 4 * FDIM), jnp.float32) for _ in range(16)],
            pltpu.VMEM((C, N_LEVELS * FDIM), jnp.float32),
            pltpu.SemaphoreType.DMA,
            pltpu.SemaphoreType.DMA,
        ],
    )
    return f(xn, tab)


# speedup vs baseline: 3.4977x; 1.0019x over previous
import math

import jax
import jax.numpy as jnp
from jax import lax
from jax.experimental import pallas as pl
from jax.experimental.pallas import tpu as pltpu
from jax.experimental.pallas import tpu_sc as plsc

N_LEVELS = 16
FDIM = 2
TSZ = 1 << 19
BASE_RES = 16
PER_LEVEL_SCALE = 1.5
P1 = 2654435761
P2 = 805459861

NC = 2
NS = 16
NW = NC * NS
LANES = 16

GRP = TSZ // 4
RES = [int(math.floor(BASE_RES * PER_LEVEL_SCALE ** l)) for l in range(N_LEVELS)]
DENSE = [(r + 1) ** 3 <= TSZ for r in RES]
CORNERS = [(cx, cy, cz) for cz in (0, 1) for cy in (0, 1) for cx in (0, 1)]

C = 512
UNROLL = 1


def _body(xn, tab, out, coords_v, frac_v, *rest):
    grp_vs = (rest[0:8], rest[8:16])
    off_vs = (rest[16:24], rest[24:32])
    rows_vs = (rest[32:40], rest[40:48])
    obuf_v = rest[48]
    sems = (rest[49], rest[50])
    n = out.shape[0]
    ppt = n // NW
    nchunk = ppt // C
    wid = lax.axis_index("s") * NC + lax.axis_index("c")
    lane = jnp.arange(LANES, dtype=jnp.int32)
    zeros16 = jnp.zeros((LANES,), jnp.int32)
    ones16 = jnp.full((LANES,), 1, jnp.int32)
    twos16 = jnp.full((LANES,), 2, jnp.int32)

    def a_and_fire(l, par):
        res = RES[l]
        f0 = 3 * par

        @plsc.parallel_loop(0, C // LANES, unroll=UNROLL)
        def pa(i, l=l, res=res, par=par, f0=f0):
            o = i * LANES
            rows16 = o + lane
            x = plsc.load_gather(coords_v, [rows16, zeros16])
            y = plsc.load_gather(coords_v, [rows16, ones16])
            z = plsc.load_gather(coords_v, [rows16, twos16])
            px = x * float(res)
            py = y * float(res)
            pz = z * float(res)
            ix = px.astype(jnp.int32)
            iy = py.astype(jnp.int32)
            iz = pz.astype(jnp.int32)
            plsc.store_scatter(frac_v, [jnp.full((LANES,), f0, jnp.int32), rows16],
                               px - ix.astype(jnp.float32))
            plsc.store_scatter(frac_v, [jnp.full((LANES,), f0 + 1, jnp.int32), rows16],
                               py - iy.astype(jnp.float32))
            plsc.store_scatter(frac_v, [jnp.full((LANES,), f0 + 2, jnp.int32), rows16],
                               pz - iz.astype(jnp.float32))
            if DENSE[l]:
                s = res + 1
                base_d = ix + s * (iy + s * iz)
                for ci, (cx, cy, cz) in enumerate(CORNERS):
                    idx = base_d + (cx + s * cy + s * s * cz)
                    plsc.store_scatter(grp_vs[par][ci], [rows16],
                                       (idx >> 2) + l * GRP)
                    plsc.store_scatter(off_vs[par][ci], [rows16], (idx & 3) << 1)
            else:
                ux = ix.astype(jnp.uint32)
                uy = iy.astype(jnp.uint32) * jnp.uint32(P1)
                uz = iz.astype(jnp.uint32) * jnp.uint32(P2)
                ux1 = ux + jnp.uint32(1)
                uy1 = uy + jnp.uint32(P1)
                uz1 = uz + jnp.uint32(P2)
                for ci, (cx, cy, cz) in enumerate(CORNERS):
                    h = (ux1 if cx else ux) ^ (uy1 if cy else uy) ^ (uz1 if cz else uz)
                    idx = (h & jnp.uint32(TSZ - 1)).astype(jnp.int32)
                    plsc.store_scatter(grp_vs[par][ci], [rows16],
                                       (idx >> 2) + l * GRP)
                    plsc.store_scatter(off_vs[par][ci], [rows16], (idx & 3) << 1)

        return [
            pltpu.async_copy(tab.at[grp_vs[par][ci]], rows_vs[par][ci], sems[par])
            for ci in range(8)
        ]

    def interp(l, par):
        f0 = 3 * par

        @plsc.parallel_loop(0, C // LANES, unroll=UNROLL)
        def pc(i, l=l, par=par, f0=f0):
            o = i * LANES
            rows16 = o + lane
            fx = plsc.load_gather(frac_v, [jnp.full((LANES,), f0, jnp.int32), rows16])
            fy = plsc.load_gather(frac_v, [jnp.full((LANES,), f0 + 1, jnp.int32), rows16])
            fz = plsc.load_gather(frac_v, [jnp.full((LANES,), f0 + 2, jnp.int32), rows16])
            gx = 1.0 - fx
            gy = 1.0 - fy
            gz = 1.0 - fz
            wxy = {(0, 0): gx * gy, (1, 0): fx * gy,
                   (0, 1): gx * fy, (1, 1): fx * fy}
            acc0 = jnp.zeros((LANES,), jnp.float32)
            acc1 = jnp.zeros((LANES,), jnp.float32)
            for ci, (cx, cy, cz) in enumerate(CORNERS):
                w = wxy[(cx, cy)] * (fz if cz else gz)
                lo = plsc.load_gather(off_vs[par][ci], [rows16])
                v0 = plsc.load_gather(rows_vs[par][ci], [rows16, lo])
                v1 = plsc.load_gather(rows_vs[par][ci], [rows16, lo + 1])
                acc0 = acc0 + v0 * w
                acc1 = acc1 + v1 * w
            plsc.store_scatter(
                obuf_v, [rows16, jnp.full((LANES,), 2 * l, jnp.int32)], acc0)
            plsc.store_scatter(
                obuf_v, [rows16, jnp.full((LANES,), 2 * l + 1, jnp.int32)], acc1)

    def chunk_body(g, carry):
        base = wid * ppt + g * C
        pltpu.sync_copy(xn.at[pl.ds(base, C)], coords_v)

        copies = a_and_fire(0, 0)
        for l in range(N_LEVELS):
            par = l % 2
            nxt = a_and_fire(l + 1, 1 - par) if l + 1 < N_LEVELS else None
            for cp in copies:
                cp.wait()
            interp(l, par)
            copies = nxt

        pltpu.sync_copy(obuf_v, out.at[pl.ds(base, C)])
        return carry

    lax.fori_loop(0, nchunk, chunk_body, 0)


def kernel(xc, table, bbox_min, bbox_max):
    n = xc.shape[0]
    assert n % (NW * C) == 0
    scale = jnp.clip(bbox_max - bbox_min, 1e-6, None)
    xn = (xc - bbox_min[None, :]) / scale[None, :]
    tabT = jnp.swapaxes(table, 1, 2).reshape(N_LEVELS, FDIM, GRP, 4)
    tab = jnp.transpose(tabT, (0, 2, 3, 1)).reshape(N_LEVELS * GRP, 4 * FDIM)

    mesh = plsc.VectorSubcoreMesh(core_axis_name="c", subcore_axis_name="s")
    f = pl.kernel(
        _body,
        out_type=jax.ShapeDtypeStruct((n, N_LEVELS * FDIM), jnp.float32),
        mesh=mesh,
        compiler_params=pltpu.CompilerParams(
            needs_layout_passes=False, use_tc_tiling_on_sc=False),
        scratch_types=[
            pltpu.VMEM((C, 3), jnp.float32),
            pltpu.VMEM((6, C), jnp.float32),
            *[pltpu.VMEM((C,), jnp.int32) for _ in range(16)],
            *[pltpu.VMEM((C,), jnp.int32) for _ in range(16)],
            *[pltpu.VMEM((C, 4 * FDIM), jnp.float32) for _ in range(16)],
            pltpu.VMEM((C, N_LEVELS * FDIM), jnp.float32),
            pltpu.SemaphoreType.DMA,
            pltpu.SemaphoreType.DMA,
        ],
    )
    return f(xn, tab)
